# baseline (speedup 1.0000x reference)
import jax
import jax.numpy as jnp
from jax import lax
from jax.experimental import pallas as pl
from jax.experimental.pallas import tpu as pltpu
from jax.experimental.pallas import tpu_sc as plsc

B = 16384
L = 50
D = 32
V = 1000000

_INFO = plsc.get_sparse_core_info()
NC = _INFO.num_cores
NS = _INFO.num_subcores
NW = NC * NS

RB = 320
RO = RB * D // 128
NCH = V // RB
CV = V * D // 128

BW = B // NW
BI = 16
NG = BW // BI
IDX_PER_IT = BI * L
GCH = 80
NGATH = IDX_PER_IT // GCH
HALF = D // 2


def _compact_body(table_hbm, comp_hbm, bufa0, bufa1, bufb, sems):
    wid = lax.axis_index("s") * NC + lax.axis_index("c")
    c_lo = (wid * NCH) >> 5
    c_hi = ((wid + 1) * NCH) >> 5
    n = c_hi - c_lo

    def start_read(c, buf, s):
        pltpu.async_copy(table_hbm.at[pl.ds(c * RB, RB)], buf, sems.at[s])

    def wait_read(buf, s):
        pltpu.make_async_copy(
            table_hbm.at[pl.ds(0, RB)], buf, sems.at[s]
        ).wait()

    def compact_and_store(c, bufa):
        def row_body(j, carry):
            r = j * 4
            for q in range(4):
                bufb[j, q * D : q * D + HALF] = bufa[r + q, 0:HALF]
                bufb[j, q * D + HALF : (q + 1) * D] = bufa[r + q, HALF:D]
            return carry

        lax.fori_loop(0, RO, row_body, 0)
        pltpu.sync_copy(bufb, comp_hbm.at[pl.ds(c * RO, RO)])

    @pl.when(n > 0)
    def _():
        start_read(c_lo, bufa0, 0)

    def chunk_body(i, _):
        c = c_lo + i

        @pl.when(lax.rem(i, 2) == 0)
        def _():
            wait_read(bufa0, 0)

            @pl.when(i + 1 < n)
            def _():
                start_read(c + 1, bufa1, 1)

            compact_and_store(c, bufa0)

        @pl.when(lax.rem(i, 2) == 1)
        def _():
            wait_read(bufa1, 1)

            @pl.when(i + 1 < n)
            def _():
                start_read(c + 1, bufa0, 0)

            compact_and_store(c, bufa1)

        return 0

    lax.fori_loop(0, n, chunk_body, 0)


def _gather_body(
    values_hbm, table_hbm, out_hbm, idx0, idx1, rows0, rows1, outb_v, sems
):
    wid = lax.axis_index("s") * NC + lax.axis_index("c")
    inv = jnp.float32(1.0 / L)
    bufs = ((idx0, rows0, 0), (idx1, rows1, 1))

    def prefetch(g, idx, rows, s):
        i0 = (wid * NG + g) * IDX_PER_IT
        pltpu.sync_copy(values_hbm.at[pl.ds(i0, IDX_PER_IT)], idx)
        for j in range(NGATH):
            pltpu.async_copy(
                table_hbm.at[idx.at[pl.ds(j * GCH, GCH)]],
                rows.at[pl.ds(j * GCH, GCH)],
                sems.at[s],
            )

    def drain(rows, s):
        pltpu.make_async_copy(
            table_hbm.at[pl.ds(0, IDX_PER_IT)], rows, sems.at[s]
        ).wait()

    def process(g, rows):
        def bag_body(b, carry):
            base = b * L
            parts0 = []
            parts1 = []
            for k in range(4):
                js = list(range(k, L, 4))
                s0 = rows[base + js[0], 0:HALF]
                s1 = rows[base + js[0], HALF:D]
                for j in js[1:]:
                    s0 = s0 + rows[base + j, 0:HALF]
                    s1 = s1 + rows[base + j, HALF:D]
                parts0.append(s0)
                parts1.append(s1)
            a0 = (parts0[0] + parts0[1]) + (parts0[2] + parts0[3])
            a1 = (parts1[0] + parts1[1]) + (parts1[2] + parts1[3])
            outb_v[b, 0:HALF] = a0 * inv
            outb_v[b, HALF:D] = a1 * inv
            return carry

        lax.fori_loop(0, BI, bag_body, 0)
        pltpu.sync_copy(outb_v, out_hbm.at[pl.ds(wid * BW + g * BI, BI)])

    prefetch(0, idx0, rows0, 0)

    def pair_body(gi, _):
        for idx, rows, s in bufs:
            g = gi * 2 + s
            nidx, nrows, ns = bufs[1 - s]

            @pl.when(g + 1 < NG)
            def _():
                prefetch(g + 1, nidx, nrows, ns)

            drain(rows, s)
            process(g, rows)
        return 0

    lax.fori_loop(0, NG // 2, pair_body, 0)


@jax.jit
def _sc_call(values, table):
    mesh = plsc.VectorSubcoreMesh(core_axis_name="c", subcore_axis_name="s")
    comp = pl.kernel(
        _compact_body,
        mesh=mesh,
        compiler_params=pltpu.CompilerParams(use_tc_tiling_on_sc=True),
        out_type=jax.ShapeDtypeStruct((CV, 128), jnp.float32),
        scratch_types=[
            pltpu.VMEM((RB, D), jnp.float32),
            pltpu.VMEM((RB, D), jnp.float32),
            pltpu.VMEM((RO, 128), jnp.float32),
            pltpu.SemaphoreType.DMA((2,)),
        ],
    )(table)
    comp2d = comp.reshape(V, D)
    return pl.kernel(
        _gather_body,
        mesh=mesh,
        compiler_params=pltpu.CompilerParams(use_tc_tiling_on_sc=False),
        out_type=jax.ShapeDtypeStruct((B, D), jnp.float32),
        scratch_types=[
            pltpu.VMEM((IDX_PER_IT,), jnp.int32),
            pltpu.VMEM((IDX_PER_IT,), jnp.int32),
            pltpu.VMEM((IDX_PER_IT, D), jnp.float32),
            pltpu.VMEM((IDX_PER_IT, D), jnp.float32),
            pltpu.VMEM((BI, D), jnp.float32),
            pltpu.SemaphoreType.DMA((2,)),
        ],
    )(values, comp2d)


def kernel(values, offsets, table):
    del offsets
    out = _sc_call(values, table)
    return out[:, None, :]


# speedup vs baseline: 1.1186x; 1.1186x over previous
import jax
import jax.numpy as jnp
from jax import lax
from jax.experimental import pallas as pl
from jax.experimental.pallas import tpu as pltpu
from jax.experimental.pallas import tpu_sc as plsc

B = 16384
L = 50
D = 32
V = 1000000

_INFO = plsc.get_sparse_core_info()
NC = _INFO.num_cores
NS = _INFO.num_subcores
NW = NC * NS

RB = 320
RO = RB * D // 128
NCH = V // RB
CV = V * D // 128

BW = B // NW
BI = 16
NG = BW // BI
IDX_PER_IT = BI * L
GCH = 80
NGATH = IDX_PER_IT // GCH
HALF = D // 2


def _compact_body(table_hbm, comp_hbm, bufa0, bufa1, bufb, sems):
    wid = lax.axis_index("s") * NC + lax.axis_index("c")
    c_lo = (wid * NCH) >> 5
    c_hi = ((wid + 1) * NCH) >> 5
    n = c_hi - c_lo

    def start_read(c, buf, s):
        pltpu.async_copy(table_hbm.at[pl.ds(c * RB, RB)], buf, sems.at[s])

    def wait_read(buf, s):
        pltpu.make_async_copy(
            table_hbm.at[pl.ds(0, RB)], buf, sems.at[s]
        ).wait()

    def compact_and_store(c, bufa):
        def row_body(j, carry):
            r = j * 4
            for q in range(4):
                bufb[j, q * D : q * D + HALF] = bufa[r + q, 0:HALF]
                bufb[j, q * D + HALF : (q + 1) * D] = bufa[r + q, HALF:D]
            return carry

        lax.fori_loop(0, RO, row_body, 0)
        pltpu.sync_copy(bufb, comp_hbm.at[pl.ds(c * RO, RO)])

    @pl.when(n > 0)
    def _():
        start_read(c_lo, bufa0, 0)

    def chunk_body(i, _):
        c = c_lo + i

        @pl.when(lax.rem(i, 2) == 0)
        def _():
            wait_read(bufa0, 0)

            @pl.when(i + 1 < n)
            def _():
                start_read(c + 1, bufa1, 1)

            compact_and_store(c, bufa0)

        @pl.when(lax.rem(i, 2) == 1)
        def _():
            wait_read(bufa1, 1)

            @pl.when(i + 1 < n)
            def _():
                start_read(c + 1, bufa0, 0)

            compact_and_store(c, bufa1)

        return 0

    lax.fori_loop(0, n, chunk_body, 0)


def _gather_body(
    values_hbm, table_hbm, out_hbm, idx0, idx1, rows0, rows1, outb_v, sems
):
    wid = lax.axis_index("s") * NC + lax.axis_index("c")
    inv = jnp.float32(1.0 / L)
    bufs = ((idx0, rows0, 0), (idx1, rows1, 1))

    def prefetch(g, idx, rows, s):
        i0 = (wid * NG + g) * IDX_PER_IT
        pltpu.sync_copy(values_hbm.at[pl.ds(i0, IDX_PER_IT)], idx)
        for j in range(NGATH):
            pltpu.async_copy(
                table_hbm.at[idx.at[pl.ds(j * GCH, GCH)]],
                rows.at[pl.ds(j * GCH, GCH)],
                sems.at[s],
            )

    def drain(rows, s):
        pltpu.make_async_copy(
            table_hbm.at[pl.ds(0, IDX_PER_IT)], rows, sems.at[s]
        ).wait()

    def process(g, rows):
        def bag_body(b, carry):
            base = b * L
            parts0 = []
            parts1 = []
            for k in range(4):
                js = list(range(k, L, 4))
                s0 = rows[base + js[0], 0:HALF]
                s1 = rows[base + js[0], HALF:D]
                for j in js[1:]:
                    s0 = s0 + rows[base + j, 0:HALF]
                    s1 = s1 + rows[base + j, HALF:D]
                parts0.append(s0)
                parts1.append(s1)
            a0 = (parts0[0] + parts0[1]) + (parts0[2] + parts0[3])
            a1 = (parts1[0] + parts1[1]) + (parts1[2] + parts1[3])
            outb_v[b, 0:HALF] = a0 * inv
            outb_v[b, HALF:D] = a1 * inv
            return carry

        lax.fori_loop(0, BI, bag_body, 0)
        pltpu.sync_copy(outb_v, out_hbm.at[pl.ds(wid * BW + g * BI, BI)])

    prefetch(0, idx0, rows0, 0)

    def pair_body(gi, _):
        for idx, rows, s in bufs:
            g = gi * 2 + s
            nidx, nrows, ns = bufs[1 - s]

            @pl.when(g + 1 < NG)
            def _():
                prefetch(g + 1, nidx, nrows, ns)

            drain(rows, s)
            process(g, rows)
        return 0

    lax.fori_loop(0, NG // 2, pair_body, 0)


@jax.jit
def _sc_call(values, comp):
    mesh = plsc.VectorSubcoreMesh(core_axis_name="c", subcore_axis_name="s")
    comp2d = comp.reshape(V, D)
    return pl.kernel(
        _gather_body,
        mesh=mesh,
        compiler_params=pltpu.CompilerParams(use_tc_tiling_on_sc=False),
        out_type=jax.ShapeDtypeStruct((B, D), jnp.float32),
        scratch_types=[
            pltpu.VMEM((IDX_PER_IT,), jnp.int32),
            pltpu.VMEM((IDX_PER_IT,), jnp.int32),
            pltpu.VMEM((IDX_PER_IT, D), jnp.float32),
            pltpu.VMEM((IDX_PER_IT, D), jnp.float32),
            pltpu.VMEM((BI, D), jnp.float32),
            pltpu.SemaphoreType.DMA((2,)),
        ],
    )(values, comp2d)


def kernel(values, offsets, table):
    del offsets
    out = _sc_call(values, table.reshape(CV, 128))
    return out[:, None, :]


# speedup vs baseline: 1.2786x; 1.1430x over previous
import jax
import jax.numpy as jnp
from jax import lax
from jax.experimental import pallas as pl
from jax.experimental.pallas import tpu as pltpu
from jax.experimental.pallas import tpu_sc as plsc

B = 16384
L = 50
D = 32
V = 1000000

_INFO = plsc.get_sparse_core_info()
NC = _INFO.num_cores
NS = _INFO.num_subcores
NW = NC * NS

W = 2048
QB = W // 4
GRID = (V + W - 1) // W
RO = W * D // 128
CV = GRID * RO
VP = CV * 128 // D

BW = B // NW
BI = 16
NG = BW // BI
IDX_PER_IT = BI * L
GCH = 80
NGATH = IDX_PER_IT // GCH
HALF = D // 2


def _tc_transpose_body(in_ref, out_ref):
    t = in_ref[...].T
    out_ref[...] = jnp.concatenate(
        [t[q * QB : (q + 1) * QB, :] for q in range(4)], axis=1
    )


def _gather_body(
    values_hbm, table_hbm, out_hbm, idx0, idx1, rows0, rows1, outb_v, sems
):
    wid = lax.axis_index("s") * NC + lax.axis_index("c")
    inv = jnp.float32(1.0 / L)
    bufs = ((idx0, rows0, 0), (idx1, rows1, 1))

    def prefetch(g, idx, rows, s):
        i0 = (wid * NG + g) * IDX_PER_IT
        pltpu.sync_copy(values_hbm.at[pl.ds(i0, IDX_PER_IT)], idx)

        def tform(k, carry):
            r = idx[pl.ds(k * 16, 16)]
            hi = r & jnp.int32(-W)
            mid = (r & jnp.int32(QB - 1)) << 2
            lo = (r >> 9) & jnp.int32(3)
            idx[pl.ds(k * 16, 16)] = hi | mid | lo
            return carry

        lax.fori_loop(0, IDX_PER_IT // 16, tform, 0)
        for j in range(NGATH):
            pltpu.async_copy(
                table_hbm.at[idx.at[pl.ds(j * GCH, GCH)]],
                rows.at[pl.ds(j * GCH, GCH)],
                sems.at[s],
            )

    def drain(rows, s):
        pltpu.make_async_copy(
            table_hbm.at[pl.ds(0, IDX_PER_IT)], rows, sems.at[s]
        ).wait()

    def process(g, rows):
        def bag_body(b, carry):
            base = b * L
            parts0 = []
            parts1 = []
            for k in range(4):
                js = list(range(k, L, 4))
                s0 = rows[base + js[0], 0:HALF]
                s1 = rows[base + js[0], HALF:D]
                for j in js[1:]:
                    s0 = s0 + rows[base + j, 0:HALF]
                    s1 = s1 + rows[base + j, HALF:D]
                parts0.append(s0)
                parts1.append(s1)
            a0 = (parts0[0] + parts0[1]) + (parts0[2] + parts0[3])
            a1 = (parts1[0] + parts1[1]) + (parts1[2] + parts1[3])
            outb_v[b, 0:HALF] = a0 * inv
            outb_v[b, HALF:D] = a1 * inv
            return carry

        lax.fori_loop(0, BI, bag_body, 0)
        pltpu.sync_copy(outb_v, out_hbm.at[pl.ds(wid * BW + g * BI, BI)])

    prefetch(0, idx0, rows0, 0)

    def pair_body(gi, _):
        for idx, rows, s in bufs:
            g = gi * 2 + s
            nidx, nrows, ns = bufs[1 - s]

            @pl.when(g + 1 < NG)
            def _():
                prefetch(g + 1, nidx, nrows, ns)

            drain(rows, s)
            process(g, rows)
        return 0

    lax.fori_loop(0, NG // 2, pair_body, 0)


@jax.jit
def _sc_call(values, tT):
    comp = pl.pallas_call(
        _tc_transpose_body,
        grid=(GRID,),
        in_specs=[pl.BlockSpec((D, W), lambda i: (0, i))],
        out_specs=pl.BlockSpec((RO, 128), lambda i: (i, 0)),
        out_shape=jax.ShapeDtypeStruct((CV, 128), jnp.float32),
    )(tT)
    comp2d = comp.reshape(VP, D)
    mesh = plsc.VectorSubcoreMesh(core_axis_name="c", subcore_axis_name="s")
    return pl.kernel(
        _gather_body,
        mesh=mesh,
        compiler_params=pltpu.CompilerParams(use_tc_tiling_on_sc=False),
        out_type=jax.ShapeDtypeStruct((B, D), jnp.float32),
        scratch_types=[
            pltpu.VMEM((IDX_PER_IT,), jnp.int32),
            pltpu.VMEM((IDX_PER_IT,), jnp.int32),
            pltpu.VMEM((IDX_PER_IT, D), jnp.float32),
            pltpu.VMEM((IDX_PER_IT, D), jnp.float32),
            pltpu.VMEM((BI, D), jnp.float32),
            pltpu.SemaphoreType.DMA((2,)),
        ],
    )(values, comp2d)


def kernel(values, offsets, table):
    del offsets
    out = _sc_call(values, table.T)
    return out[:, None, :]


# speedup vs baseline: 1.8836x; 1.4731x over previous
import jax
import jax.numpy as jnp
from jax import lax
from jax.experimental import pallas as pl
from jax.experimental.pallas import tpu as pltpu
from jax.experimental.pallas import tpu_sc as plsc

B = 16384
L = 50
D = 32
V = 1000000

_INFO = plsc.get_sparse_core_info()
NC = _INFO.num_cores
NS = _INFO.num_subcores
NW = NC * NS

W = 8192
QB = W // 4
SHQ = QB.bit_length() - 1
GRID = (V + W - 1) // W
RO = W * D // 128
CV = GRID * RO
VP = GRID * W

BW = B // NW
BI = 16
NG = BW // BI
IDX_PER_IT = BI * L
GCH = 80
NGATH = IDX_PER_IT // GCH
HALF = D // 2


def _tc_transpose_body(in_ref, out_ref):
    t = in_ref[...].T
    out_ref[...] = jnp.concatenate(
        [t[q * QB : (q + 1) * QB, :] for q in range(4)], axis=1
    )


def _gather_body(
    values_hbm, table_hbm, out_hbm, idx0, idx1, rows0, rows1, outb_v, sems
):
    wid = lax.axis_index("s") * NC + lax.axis_index("c")
    inv = jnp.float32(1.0 / L)
    bufs = ((idx0, rows0, 0), (idx1, rows1, 1))

    def prefetch(g, idx, rows, s):
        i0 = (wid * NG + g) * IDX_PER_IT
        pltpu.sync_copy(values_hbm.at[pl.ds(i0, IDX_PER_IT)], idx)

        def tform(k, carry):
            r = idx[pl.ds(k * 16, 16)]
            hi = r & jnp.int32(-W)
            mid = (r & jnp.int32(QB - 1)) << 2
            lo = (r >> SHQ) & jnp.int32(3)
            idx[pl.ds(k * 16, 16)] = hi | mid | lo
            return carry

        lax.fori_loop(0, IDX_PER_IT // 16, tform, 0)
        for j in range(NGATH):
            pltpu.async_copy(
                table_hbm.at[idx.at[pl.ds(j * GCH, GCH)]],
                rows.at[pl.ds(j * GCH, GCH)],
                sems.at[s],
            )

    def drain(rows, s):
        pltpu.make_async_copy(
            table_hbm.at[pl.ds(0, IDX_PER_IT)], rows, sems.at[s]
        ).wait()

    def process(g, rows):
        def bag_body(b, carry):
            base = b * L
            parts0 = []
            parts1 = []
            for k in range(4):
                js = list(range(k, L, 4))
                s0 = rows[base + js[0], 0:HALF]
                s1 = rows[base + js[0], HALF:D]
                for j in js[1:]:
                    s0 = s0 + rows[base + j, 0:HALF]
                    s1 = s1 + rows[base + j, HALF:D]
                parts0.append(s0)
                parts1.append(s1)
            a0 = (parts0[0] + parts0[1]) + (parts0[2] + parts0[3])
            a1 = (parts1[0] + parts1[1]) + (parts1[2] + parts1[3])
            outb_v[b, 0:HALF] = a0 * inv
            outb_v[b, HALF:D] = a1 * inv
            return carry

        lax.fori_loop(0, BI, bag_body, 0)
        pltpu.sync_copy(outb_v, out_hbm.at[pl.ds(wid * BW + g * BI, BI)])

    prefetch(0, idx0, rows0, 0)

    def pair_body(gi, _):
        for idx, rows, s in bufs:
            g = gi * 2 + s
            nidx, nrows, ns = bufs[1 - s]

            @pl.when(g + 1 < NG)
            def _():
                prefetch(g + 1, nidx, nrows, ns)

            drain(rows, s)
            process(g, rows)
        return 0

    lax.fori_loop(0, NG // 2, pair_body, 0)


@jax.jit
def _sc_call(values, tT):
    comp = pl.pallas_call(
        _tc_transpose_body,
        grid=(GRID,),
        in_specs=[pl.BlockSpec((D, W), lambda i: (0, i))],
        out_specs=pl.BlockSpec((RO, 128), lambda i: (i, 0)),
        out_shape=jax.ShapeDtypeStruct((CV, 128), jnp.float32),
    )(tT)
    comp2d = comp.reshape(VP, D)
    mesh = plsc.VectorSubcoreMesh(core_axis_name="c", subcore_axis_name="s")
    return pl.kernel(
        _gather_body,
        mesh=mesh,
        compiler_params=pltpu.CompilerParams(use_tc_tiling_on_sc=False),
        out_type=jax.ShapeDtypeStruct((B, D), jnp.float32),
        scratch_types=[
            pltpu.VMEM((IDX_PER_IT,), jnp.int32),
            pltpu.VMEM((IDX_PER_IT,), jnp.int32),
            pltpu.VMEM((IDX_PER_IT, D), jnp.float32),
            pltpu.VMEM((IDX_PER_IT, D), jnp.float32),
            pltpu.VMEM((BI, D), jnp.float32),
            pltpu.SemaphoreType.DMA((2,)),
        ],
    )(values, comp2d)


def kernel(values, offsets, table):
    del offsets
    out = _sc_call(values, table.T)
    return out[:, None, :]


# speedup vs baseline: 1.9063x; 1.0121x over previous
import jax
import jax.numpy as jnp
from jax import lax
from jax.experimental import pallas as pl
from jax.experimental.pallas import tpu as pltpu
from jax.experimental.pallas import tpu_sc as plsc

B = 16384
L = 50
D = 32
V = 1000000

_INFO = plsc.get_sparse_core_info()
NC = _INFO.num_cores
NS = _INFO.num_subcores
NW = NC * NS

W = 16384
QB = W // 4
SHQ = QB.bit_length() - 1
GRID = (V + W - 1) // W
RO = W * D // 128
CV = GRID * RO
VP = GRID * W

BW = B // NW
BI = 16
NG = BW // BI
IDX_PER_IT = BI * L
GCH = 80
NGATH = IDX_PER_IT // GCH
HALF = D // 2


def _tc_transpose_body(in_ref, out_ref):
    t = in_ref[...].T
    out_ref[...] = jnp.concatenate(
        [t[q * QB : (q + 1) * QB, :] for q in range(4)], axis=1
    )


def _gather_body(
    values_hbm, table_hbm, out_hbm, idx0, idx1, rows0, rows1, outb_v, sems
):
    wid = lax.axis_index("s") * NC + lax.axis_index("c")
    inv = jnp.float32(1.0 / L)
    bufs = ((idx0, rows0, 0), (idx1, rows1, 1))

    def prefetch(g, idx, rows, s):
        i0 = (wid * NG + g) * IDX_PER_IT
        pltpu.sync_copy(values_hbm.at[pl.ds(i0, IDX_PER_IT)], idx)

        def tform(k, carry):
            r = idx[pl.ds(k * 16, 16)]
            hi = r & jnp.int32(-W)
            mid = (r & jnp.int32(QB - 1)) << 2
            lo = (r >> SHQ) & jnp.int32(3)
            idx[pl.ds(k * 16, 16)] = hi | mid | lo
            return carry

        lax.fori_loop(0, IDX_PER_IT // 16, tform, 0)
        for j in range(NGATH):
            pltpu.async_copy(
                table_hbm.at[idx.at[pl.ds(j * GCH, GCH)]],
                rows.at[pl.ds(j * GCH, GCH)],
                sems.at[s],
            )

    def drain(rows, s):
        pltpu.make_async_copy(
            table_hbm.at[pl.ds(0, IDX_PER_IT)], rows, sems.at[s]
        ).wait()

    def process(g, rows):
        def bag_body(b, carry):
            base = b * L
            parts0 = []
            parts1 = []
            for k in range(4):
                js = list(range(k, L, 4))
                s0 = rows[base + js[0], 0:HALF]
                s1 = rows[base + js[0], HALF:D]
                for j in js[1:]:
                    s0 = s0 + rows[base + j, 0:HALF]
                    s1 = s1 + rows[base + j, HALF:D]
                parts0.append(s0)
                parts1.append(s1)
            a0 = (parts0[0] + parts0[1]) + (parts0[2] + parts0[3])
            a1 = (parts1[0] + parts1[1]) + (parts1[2] + parts1[3])
            outb_v[b, 0:HALF] = a0 * inv
            outb_v[b, HALF:D] = a1 * inv
            return carry

        lax.fori_loop(0, BI, bag_body, 0)
        pltpu.sync_copy(outb_v, out_hbm.at[pl.ds(wid * BW + g * BI, BI)])

    prefetch(0, idx0, rows0, 0)

    def pair_body(gi, _):
        for idx, rows, s in bufs:
            g = gi * 2 + s
            nidx, nrows, ns = bufs[1 - s]

            @pl.when(g + 1 < NG)
            def _():
                prefetch(g + 1, nidx, nrows, ns)

            drain(rows, s)
            process(g, rows)
        return 0

    lax.fori_loop(0, NG // 2, pair_body, 0)


@jax.jit
def _sc_call(values, tT):
    comp = pl.pallas_call(
        _tc_transpose_body,
        grid=(GRID,),
        in_specs=[pl.BlockSpec((D, W), lambda i: (0, i))],
        out_specs=pl.BlockSpec((RO, 128), lambda i: (i, 0)),
        out_shape=jax.ShapeDtypeStruct((CV, 128), jnp.float32),
    )(tT)
    comp2d = comp.reshape(VP, D)
    mesh = plsc.VectorSubcoreMesh(core_axis_name="c", subcore_axis_name="s")
    return pl.kernel(
        _gather_body,
        mesh=mesh,
        compiler_params=pltpu.CompilerParams(use_tc_tiling_on_sc=False),
        out_type=jax.ShapeDtypeStruct((B, D), jnp.float32),
        scratch_types=[
            pltpu.VMEM((IDX_PER_IT,), jnp.int32),
            pltpu.VMEM((IDX_PER_IT,), jnp.int32),
            pltpu.VMEM((IDX_PER_IT, D), jnp.float32),
            pltpu.VMEM((IDX_PER_IT, D), jnp.float32),
            pltpu.VMEM((BI, D), jnp.float32),
            pltpu.SemaphoreType.DMA((2,)),
        ],
    )(values, comp2d)


def kernel(values, offsets, table):
    del offsets
    out = _sc_call(values, table.T)
    return out[:, None, :]


# speedup vs baseline: 1.9553x; 1.0257x over previous
import jax
import jax.numpy as jnp
from jax import lax
from jax.experimental import pallas as pl
from jax.experimental.pallas import tpu as pltpu
from jax.experimental.pallas import tpu_sc as plsc

B = 16384
L = 50
D = 32
V = 1000000

_INFO = plsc.get_sparse_core_info()
NC = _INFO.num_cores
NS = _INFO.num_subcores
NW = NC * NS

W = 16384
QB = W // 4
SHQ = QB.bit_length() - 1
GRID = (V + W - 1) // W
RO = W * D // 128
CV = GRID * RO
VP = GRID * W

BW = B // NW
BI = 32
NG = BW // BI
IDX_PER_IT = BI * L
GCH = 80
NGATH = IDX_PER_IT // GCH
HALF = D // 2


def _tc_transpose_body(in_ref, out_ref):
    t = in_ref[...].T
    out_ref[...] = jnp.concatenate(
        [t[q * QB : (q + 1) * QB, :] for q in range(4)], axis=1
    )


def _gather_body(
    values_hbm, table_hbm, out_hbm, idx0, idx1, rows0, rows1, outb_v, sems
):
    wid = lax.axis_index("s") * NC + lax.axis_index("c")
    inv = jnp.float32(1.0 / L)
    bufs = ((idx0, rows0, 0), (idx1, rows1, 1))

    def prefetch(g, idx, rows, s):
        i0 = (wid * NG + g) * IDX_PER_IT
        pltpu.sync_copy(values_hbm.at[pl.ds(i0, IDX_PER_IT)], idx)

        def tform(k, carry):
            r = idx[pl.ds(k * 16, 16)]
            hi = r & jnp.int32(-W)
            mid = (r & jnp.int32(QB - 1)) << 2
            lo = (r >> SHQ) & jnp.int32(3)
            idx[pl.ds(k * 16, 16)] = hi | mid | lo
            return carry

        lax.fori_loop(0, IDX_PER_IT // 16, tform, 0)
        for j in range(NGATH):
            pltpu.async_copy(
                table_hbm.at[idx.at[pl.ds(j * GCH, GCH)]],
                rows.at[pl.ds(j * GCH, GCH)],
                sems.at[s],
            )

    def drain(rows, s):
        pltpu.make_async_copy(
            table_hbm.at[pl.ds(0, IDX_PER_IT)], rows, sems.at[s]
        ).wait()

    def process(g, rows):
        def bag_body(b, carry):
            base = b * L
            parts0 = []
            parts1 = []
            for k in range(4):
                js = list(range(k, L, 4))
                s0 = rows[base + js[0], 0:HALF]
                s1 = rows[base + js[0], HALF:D]
                for j in js[1:]:
                    s0 = s0 + rows[base + j, 0:HALF]
                    s1 = s1 + rows[base + j, HALF:D]
                parts0.append(s0)
                parts1.append(s1)
            a0 = (parts0[0] + parts0[1]) + (parts0[2] + parts0[3])
            a1 = (parts1[0] + parts1[1]) + (parts1[2] + parts1[3])
            outb_v[b, 0:HALF] = a0 * inv
            outb_v[b, HALF:D] = a1 * inv
            return carry

        lax.fori_loop(0, BI, bag_body, 0)
        pltpu.sync_copy(outb_v, out_hbm.at[pl.ds(wid * BW + g * BI, BI)])

    prefetch(0, idx0, rows0, 0)

    def pair_body(gi, _):
        for idx, rows, s in bufs:
            g = gi * 2 + s
            nidx, nrows, ns = bufs[1 - s]

            @pl.when(g + 1 < NG)
            def _():
                prefetch(g + 1, nidx, nrows, ns)

            drain(rows, s)
            process(g, rows)
        return 0

    lax.fori_loop(0, NG // 2, pair_body, 0)


@jax.jit
def _sc_call(values, tT):
    comp = pl.pallas_call(
        _tc_transpose_body,
        grid=(GRID,),
        in_specs=[pl.BlockSpec((D, W), lambda i: (0, i))],
        out_specs=pl.BlockSpec((RO, 128), lambda i: (i, 0)),
        out_shape=jax.ShapeDtypeStruct((CV, 128), jnp.float32),
    )(tT)
    comp2d = comp.reshape(VP, D)
    mesh = plsc.VectorSubcoreMesh(core_axis_name="c", subcore_axis_name="s")
    return pl.kernel(
        _gather_body,
        mesh=mesh,
        compiler_params=pltpu.CompilerParams(use_tc_tiling_on_sc=False),
        out_type=jax.ShapeDtypeStruct((B, D), jnp.float32),
        scratch_types=[
            pltpu.VMEM((IDX_PER_IT,), jnp.int32),
            pltpu.VMEM((IDX_PER_IT,), jnp.int32),
            pltpu.VMEM((IDX_PER_IT, D), jnp.float32),
            pltpu.VMEM((IDX_PER_IT, D), jnp.float32),
            pltpu.VMEM((BI, D), jnp.float32),
            pltpu.SemaphoreType.DMA((2,)),
        ],
    )(values, comp2d)


def kernel(values, offsets, table):
    del offsets
    out = _sc_call(values, table.T)
    return out[:, None, :]


# speedup vs baseline: 1.9692x; 1.0071x over previous
import jax
import jax.numpy as jnp
from jax import lax
from jax.experimental import pallas as pl
from jax.experimental.pallas import tpu as pltpu
from jax.experimental.pallas import tpu_sc as plsc

B = 16384
L = 50
D = 32
V = 1000000

_INFO = plsc.get_sparse_core_info()
NC = _INFO.num_cores
NS = _INFO.num_subcores
NW = NC * NS

W = 32768
QB = W // 4
SHQ = QB.bit_length() - 1
GRID = (V + W - 1) // W
RO = W * D // 128
CV = GRID * RO
VP = GRID * W

BW = B // NW
BI = 32
NG = BW // BI
IDX_PER_IT = BI * L
GCH = 80
NGATH = IDX_PER_IT // GCH
HALF = D // 2


def _tc_transpose_body(in_ref, out_ref):
    t = in_ref[...].T
    out_ref[...] = jnp.concatenate(
        [t[q * QB : (q + 1) * QB, :] for q in range(4)], axis=1
    )


def _gather_body(
    values_hbm, table_hbm, out_hbm, idx0, idx1, rows0, rows1, outb_v, sems
):
    wid = lax.axis_index("s") * NC + lax.axis_index("c")
    inv = jnp.float32(1.0 / L)
    bufs = ((idx0, rows0, 0), (idx1, rows1, 1))

    def prefetch(g, idx, rows, s):
        i0 = (wid * NG + g) * IDX_PER_IT
        pltpu.sync_copy(values_hbm.at[pl.ds(i0, IDX_PER_IT)], idx)

        def tform(k, carry):
            r = idx[pl.ds(k * 16, 16)]
            hi = r & jnp.int32(-W)
            mid = (r & jnp.int32(QB - 1)) << 2
            lo = (r >> SHQ) & jnp.int32(3)
            idx[pl.ds(k * 16, 16)] = hi | mid | lo
            return carry

        lax.fori_loop(0, IDX_PER_IT // 16, tform, 0)
        for j in range(NGATH):
            pltpu.async_copy(
                table_hbm.at[idx.at[pl.ds(j * GCH, GCH)]],
                rows.at[pl.ds(j * GCH, GCH)],
                sems.at[s],
            )

    def drain(rows, s):
        pltpu.make_async_copy(
            table_hbm.at[pl.ds(0, IDX_PER_IT)], rows, sems.at[s]
        ).wait()

    def process(g, rows):
        def bag_body(b, carry):
            base = b * L
            parts0 = []
            parts1 = []
            for k in range(4):
                js = list(range(k, L, 4))
                s0 = rows[base + js[0], 0:HALF]
                s1 = rows[base + js[0], HALF:D]
                for j in js[1:]:
                    s0 = s0 + rows[base + j, 0:HALF]
                    s1 = s1 + rows[base + j, HALF:D]
                parts0.append(s0)
                parts1.append(s1)
            a0 = (parts0[0] + parts0[1]) + (parts0[2] + parts0[3])
            a1 = (parts1[0] + parts1[1]) + (parts1[2] + parts1[3])
            outb_v[b, 0:HALF] = a0 * inv
            outb_v[b, HALF:D] = a1 * inv
            return carry

        lax.fori_loop(0, BI, bag_body, 0)
        pltpu.sync_copy(outb_v, out_hbm.at[pl.ds(wid * BW + g * BI, BI)])

    prefetch(0, idx0, rows0, 0)

    def pair_body(gi, _):
        for idx, rows, s in bufs:
            g = gi * 2 + s
            nidx, nrows, ns = bufs[1 - s]

            @pl.when(g + 1 < NG)
            def _():
                prefetch(g + 1, nidx, nrows, ns)

            drain(rows, s)
            process(g, rows)
        return 0

    lax.fori_loop(0, NG // 2, pair_body, 0)


@jax.jit
def _sc_call(values, tT):
    comp = pl.pallas_call(
        _tc_transpose_body,
        grid=(GRID,),
        in_specs=[pl.BlockSpec((D, W), lambda i: (0, i))],
        out_specs=pl.BlockSpec((RO, 128), lambda i: (i, 0)),
        out_shape=jax.ShapeDtypeStruct((CV, 128), jnp.float32),
    )(tT)
    comp2d = comp.reshape(VP, D)
    mesh = plsc.VectorSubcoreMesh(core_axis_name="c", subcore_axis_name="s")
    return pl.kernel(
        _gather_body,
        mesh=mesh,
        compiler_params=pltpu.CompilerParams(use_tc_tiling_on_sc=False),
        out_type=jax.ShapeDtypeStruct((B, D), jnp.float32),
        scratch_types=[
            pltpu.VMEM((IDX_PER_IT,), jnp.int32),
            pltpu.VMEM((IDX_PER_IT,), jnp.int32),
            pltpu.VMEM((IDX_PER_IT, D), jnp.float32),
            pltpu.VMEM((IDX_PER_IT, D), jnp.float32),
            pltpu.VMEM((BI, D), jnp.float32),
            pltpu.SemaphoreType.DMA((2,)),
        ],
    )(values, comp2d)


def kernel(values, offsets, table):
    del offsets
    out = _sc_call(values, table.T)
    return out[:, None, :]
